# Initial kernel scaffold; baseline (speedup 1.0000x reference)
import jax
import jax.numpy as jnp
from jax.experimental import pallas as pl


def kernel(embeddings, antonym_pairs):
    raise NotImplementedError("write your pallas kernel here")


# speedup vs baseline: 1.1273x; 1.1273x over previous
import jax
import jax.numpy as jnp
from jax import lax
from jax.experimental import pallas as pl
from jax.experimental.pallas import tpu as pltpu
from jax.experimental.pallas import tpu_sc as plsc
import functools

_D = 128
_P = 4096
_LANES = 16


@functools.lru_cache(maxsize=None)
def _build_sc_kernel(num_pairs, dim):
    info = plsc.get_sparse_core_info()
    nc, ns = info.num_cores, info.num_subcores
    nw = nc * ns
    ppw = num_pairs // nw
    dvec = dim // _LANES

    mesh = plsc.VectorSubcoreMesh(core_axis_name="c", subcore_axis_name="s")

    @functools.partial(
        pl.kernel,
        out_type=jax.ShapeDtypeStruct((nw, _LANES), jnp.float32),
        mesh=mesh,
        scratch_types=[
            pltpu.VMEM((2, ppw), jnp.int32),
            pltpu.VMEM((2, ppw, dim), jnp.float32),
            pltpu.VMEM((_LANES,), jnp.float32),
            pltpu.SemaphoreType.DMA,
        ],
    )
    def polarity_kernel(table_hbm, idx_hbm, out_hbm, idx_v, rows_v, acc_v, sem):
        wid = lax.axis_index("s") * nc + lax.axis_index("c")

        pltpu.sync_copy(idx_hbm.at[wid], idx_v)
        cp0 = pltpu.async_copy(table_hbm.at[idx_v.at[0]], rows_v.at[0], sem)
        cp1 = pltpu.async_copy(table_hbm.at[idx_v.at[1]], rows_v.at[1], sem)
        cp0.wait()
        cp1.wait()

        half = jnp.full((_LANES,), 0.5, jnp.float32)
        one = jnp.full((_LANES,), 1.0, jnp.float32)
        tenth = jnp.full((_LANES,), 0.1, jnp.float32)
        zero = jnp.zeros((_LANES,), jnp.float32)

        def pair_body(q, acc):
            for d in range(dvec):
                a = rows_v[0, q, pl.ds(d * _LANES, _LANES)]
                b = rows_v[1, q, pl.ds(d * _LANES, _LANES)]
                opposite = (a < zero) ^ (b < zero)
                any_zero = (a == zero) | (b == zero)
                abs_sum = jnp.abs(a) + jnp.abs(b)
                factor = jnp.where(opposite, -half, one)
                acc = acc + jnp.where(any_zero, tenth, factor * abs_sum)
            return acc

        acc = lax.fori_loop(0, ppw, pair_body, zero)
        acc_v[...] = acc
        pltpu.sync_copy(acc_v, out_hbm.at[wid])

    return polarity_kernel, nw, ppw


def kernel(embeddings, antonym_pairs):
    num_pairs, dim = antonym_pairs.shape[0], embeddings.shape[1]
    sc_kernel, nw, ppw = _build_sc_kernel(num_pairs, dim)
    idx = jnp.transpose(antonym_pairs.astype(jnp.int32)).reshape(2, nw, ppw)
    idx = jnp.transpose(idx, (1, 0, 2))
    partials = sc_kernel(embeddings, idx)
    return partials.sum() * jnp.float32(1.0 / num_pairs)


# speedup vs baseline: 1.3358x; 1.1850x over previous
import jax
import jax.numpy as jnp
from jax import lax
from jax.experimental import pallas as pl
from jax.experimental.pallas import tpu as pltpu
from jax.experimental.pallas import tpu_sc as plsc
import functools

_D = 128
_P = 4096
_LANES = 16


@functools.lru_cache(maxsize=None)
def _build_sc_kernel(num_pairs, dim):
    info = plsc.get_sparse_core_info()
    nc, ns = info.num_cores, info.num_subcores
    nw = nc * ns
    ppw = num_pairs // nw
    dvec = dim // _LANES

    mesh = plsc.VectorSubcoreMesh(core_axis_name="c", subcore_axis_name="s")
    nchunks = 4
    cpw = ppw // nchunks

    @functools.partial(
        pl.kernel,
        out_type=jax.ShapeDtypeStruct((nw, _LANES), jnp.float32),
        mesh=mesh,
        scratch_types=[
            pltpu.VMEM((2, ppw), jnp.int32),
            pltpu.VMEM((2, ppw, dim), jnp.float32),
            pltpu.VMEM((_LANES,), jnp.float32),
            pltpu.SemaphoreType.DMA,
        ],
    )
    def polarity_kernel(table_hbm, idx_hbm, out_hbm, idx_v, rows_v, acc_v, sem):
        wid = lax.axis_index("s") * nc + lax.axis_index("c")

        pltpu.sync_copy(idx_hbm.at[wid], idx_v)
        copies = []
        for c in range(nchunks):
            sl = pl.ds(c * cpw, cpw)
            copies.append((
                pltpu.async_copy(
                    table_hbm.at[idx_v.at[0, sl]], rows_v.at[0, sl], sem),
                pltpu.async_copy(
                    table_hbm.at[idx_v.at[1, sl]], rows_v.at[1, sl], sem),
            ))

        half = jnp.full((_LANES,), 0.5, jnp.float32)
        one = jnp.full((_LANES,), 1.0, jnp.float32)
        tenth = jnp.full((_LANES,), 0.1, jnp.float32)
        zero = jnp.zeros((_LANES,), jnp.float32)

        def pair_body(q, acc):
            for d in range(dvec):
                a = rows_v[0, q, pl.ds(d * _LANES, _LANES)]
                b = rows_v[1, q, pl.ds(d * _LANES, _LANES)]
                opposite = (a < zero) ^ (b < zero)
                aa = jnp.abs(a)
                ab = jnp.abs(b)
                any_zero = jnp.minimum(aa, ab) == zero
                abs_sum = aa + ab
                factor = jnp.where(opposite, -half, one)
                acc = acc + jnp.where(any_zero, tenth, factor * abs_sum)
            return acc

        acc = zero
        for c in range(nchunks):
            copies[c][0].wait()
            copies[c][1].wait()
            acc = lax.fori_loop(c * cpw, (c + 1) * cpw, pair_body, acc)
        acc_v[...] = acc
        pltpu.sync_copy(acc_v, out_hbm.at[wid])

    return polarity_kernel, nw, ppw


def kernel(embeddings, antonym_pairs):
    num_pairs, dim = antonym_pairs.shape[0], embeddings.shape[1]
    sc_kernel, nw, ppw = _build_sc_kernel(num_pairs, dim)
    idx = jnp.transpose(antonym_pairs.astype(jnp.int32)).reshape(2, nw, ppw)
    idx = jnp.transpose(idx, (1, 0, 2))
    partials = sc_kernel(embeddings, idx)
    return partials.sum() * jnp.float32(1.0 / num_pairs)
